# Initial kernel scaffold; baseline (speedup 1.0000x reference)
import jax
import jax.numpy as jnp
from jax.experimental import pallas as pl


def kernel(node_features, edge_weights, edge_index, W, b):
    raise NotImplementedError("write your pallas kernel here")


# speedup vs baseline: 19.0342x; 19.0342x over previous
import jax
import jax.numpy as jnp
from jax import lax
from jax.experimental import pallas as pl
from jax.experimental.pallas import tpu as pltpu
from jax.experimental.pallas import tpu_sc as plsc

N_NODES = 10000
N_EDGES = 320000
T_DIM = 12
F_DIM = 16
T_HALF = T_DIM // 2
HROW = T_HALF * F_DIM

NUM_CORES = 2
NUM_SUBCORES = 16

CHUNK = 80
EDGES_PER_TILE = N_EDGES // NUM_SUBCORES
CHUNKS_PER_TILE = EDGES_PER_TILE // CHUNK

STRIPE = 624
TAIL_ROWS = N_NODES - NUM_SUBCORES * STRIPE
ZROWS = 16


def _sc_body(x2_hbm, src_hbm, dst_hbm, w_hbm, p0_hbm, p1_hbm,
             sidx, gidx, didx, wbuf, rows, zbuf, acc, gsem):
    c = lax.axis_index("c")
    s = lax.axis_index("s")

    def _zero_row(r, _):
        for j in range(T_HALF):
            zbuf[r, pl.ds(j * F_DIM, F_DIM)] = jnp.zeros((F_DIM,), jnp.float32)
        return 0
    lax.fori_loop(0, ZROWS, _zero_row, 0)
    row0 = s * STRIPE

    def _zfill(z, _):
        pltpu.sync_copy(zbuf, acc.at[pl.ds(row0 + z * ZROWS, ZROWS)])
        return 0
    lax.fori_loop(0, STRIPE // ZROWS, _zfill, 0)

    @pl.when(s == 0)
    def _():
        pltpu.sync_copy(zbuf, acc.at[pl.ds(NUM_SUBCORES * STRIPE, TAIL_ROWS)])
    plsc.subcore_barrier()

    e_base = s * EDGES_PER_TILE

    def _chunk(k, _):
        e0 = e_base + k * CHUNK
        pltpu.sync_copy(src_hbm.at[pl.ds(e0, CHUNK)], sidx)
        pltpu.sync_copy(dst_hbm.at[pl.ds(e0, CHUNK)], didx)
        pltpu.sync_copy(w_hbm.at[pl.ds(e0 * F_DIM, CHUNK * F_DIM)], wbuf)
        for j in range(CHUNK // F_DIM):
            sl = pl.ds(j * F_DIM, F_DIM)
            gidx[sl] = sidx[sl] * 2 + c
        pltpu.async_copy(x2_hbm.at[gidx], rows, gsem).wait()

        def _edge(e, _):
            wv = wbuf[pl.ds(e * F_DIM + c * 8, F_DIM)]
            for t in range(T_HALF):
                rows[e, pl.ds(t * F_DIM, F_DIM)] = (
                    rows[e, pl.ds(t * F_DIM, F_DIM)] * wv[t])
            return 0
        lax.fori_loop(0, CHUNK, _edge, 0)

        pltpu.sync_copy(rows, acc.at[didx], add=True)
        return 0

    lax.fori_loop(0, CHUNKS_PER_TILE, _chunk, 0)
    plsc.subcore_barrier()

    tail0 = NUM_SUBCORES * STRIPE

    @pl.when(c == 0)
    def _():
        pltpu.sync_copy(acc.at[pl.ds(row0, STRIPE)],
                        p0_hbm.at[pl.ds(row0, STRIPE)])

        @pl.when(s == 0)
        def _():
            pltpu.sync_copy(acc.at[pl.ds(tail0, TAIL_ROWS)],
                            p0_hbm.at[pl.ds(tail0, TAIL_ROWS)])

    @pl.when(c == 1)
    def _():
        pltpu.sync_copy(acc.at[pl.ds(row0, STRIPE)],
                        p1_hbm.at[pl.ds(row0, STRIPE)])

        @pl.when(s == 0)
        def _():
            pltpu.sync_copy(acc.at[pl.ds(tail0, TAIL_ROWS)],
                            p1_hbm.at[pl.ds(tail0, TAIL_ROWS)])


def _sc_aggregate(x2, src, dst, wpad):
    mesh = plsc.VectorSubcoreMesh(
        core_axis_name="c", subcore_axis_name="s",
        num_cores=NUM_CORES, num_subcores=NUM_SUBCORES)
    f = pl.kernel(
        _sc_body,
        out_type=(
            jax.ShapeDtypeStruct((N_NODES, HROW), jnp.float32),
            jax.ShapeDtypeStruct((N_NODES, HROW), jnp.float32),
        ),
        mesh=mesh,
        scratch_types=[
            pltpu.VMEM((CHUNK,), jnp.int32),
            pltpu.VMEM((CHUNK,), jnp.int32),
            pltpu.VMEM((CHUNK,), jnp.int32),
            pltpu.VMEM((CHUNK * F_DIM,), jnp.float32),
            pltpu.VMEM((CHUNK, HROW), jnp.float32),
            pltpu.VMEM((ZROWS, HROW), jnp.float32),
            pltpu.VMEM_SHARED((N_NODES, HROW), jnp.float32),
            pltpu.SemaphoreType.DMA,
        ],
        compiler_params=pltpu.CompilerParams(use_tc_tiling_on_sc=False),
    )
    return f(x2, src, dst, wpad)


def _tc_linear_body(p0_ref, p1_ref, wt_ref, b_ref, o0_ref, o1_ref):
    wt = wt_ref[...]
    b = b_ref[...]
    o0_ref[...] = (
        jnp.dot(p0_ref[...], wt, preferred_element_type=jnp.float32) + b)
    o1_ref[...] = (
        jnp.dot(p1_ref[...], wt, preferred_element_type=jnp.float32) + b)


def _tc_linear(p0r, p1r, wt, b2):
    rows_total = N_NODES * T_HALF
    blk = 6000
    grid = rows_total // blk
    return pl.pallas_call(
        _tc_linear_body,
        grid=(grid,),
        in_specs=[
            pl.BlockSpec((blk, F_DIM), lambda i: (i, 0)),
            pl.BlockSpec((blk, F_DIM), lambda i: (i, 0)),
            pl.BlockSpec((F_DIM, F_DIM), lambda i: (0, 0)),
            pl.BlockSpec((1, F_DIM), lambda i: (0, 0)),
        ],
        out_specs=[
            pl.BlockSpec((blk, F_DIM), lambda i: (i, 0)),
            pl.BlockSpec((blk, F_DIM), lambda i: (i, 0)),
        ],
        out_shape=[
            jax.ShapeDtypeStruct((rows_total, F_DIM), jnp.float32),
            jax.ShapeDtypeStruct((rows_total, F_DIM), jnp.float32),
        ],
    )(p0r, p1r, wt, b2)


def kernel(node_features, edge_weights, edge_index, W, b):
    x2 = node_features.reshape(N_NODES * 2, HROW)
    src = edge_index[0]
    dst = edge_index[1]
    wpad = jnp.pad(
        edge_weights.T.reshape(N_EDGES, 2, T_HALF),
        ((0, 0), (0, 0), (0, 8 - T_HALF))).reshape(N_EDGES * F_DIM)
    p0, p1 = _sc_aggregate(x2, src, dst, wpad)
    o0, o1 = _tc_linear(
        p0.reshape(N_NODES * T_HALF, F_DIM),
        p1.reshape(N_NODES * T_HALF, F_DIM),
        W.T, b.reshape(1, F_DIM))
    out = jnp.concatenate(
        [o0.reshape(N_NODES, T_HALF, F_DIM),
         o1.reshape(N_NODES, T_HALF, F_DIM)], axis=1)
    return out


# speedup vs baseline: 25.3503x; 1.3318x over previous
import jax
import jax.numpy as jnp
from jax import lax
from jax.experimental import pallas as pl
from jax.experimental.pallas import tpu as pltpu
from jax.experimental.pallas import tpu_sc as plsc

N_NODES = 10000
N_EDGES = 320000
T_DIM = 12
F_DIM = 16
T_HALF = T_DIM // 2
HROW = T_HALF * F_DIM

NUM_CORES = 2
NUM_SUBCORES = 16

CHUNK = 80
EDGES_PER_TILE = N_EDGES // NUM_SUBCORES
CHUNKS_PER_TILE = EDGES_PER_TILE // CHUNK
NBUF = 5
GROUPS_PER_TILE = CHUNKS_PER_TILE // NBUF

STRIPE = 624
TAIL_ROWS = N_NODES - NUM_SUBCORES * STRIPE
ZROWS = 16


def _sc_body(x2_hbm, src_hbm, dst_hbm, w_hbm, p0_hbm, p1_hbm,
             sidx, gidx, didx, wbuf, rows, zbuf, acc, *sems):
    c = lax.axis_index("c")
    s = lax.axis_index("s")

    def _zero_row(r, _):
        for j in range(T_HALF):
            zbuf[r, pl.ds(j * F_DIM, F_DIM)] = jnp.zeros((F_DIM,), jnp.float32)
        return 0
    lax.fori_loop(0, ZROWS, _zero_row, 0)
    row0 = s * STRIPE

    def _zfill(z, _):
        pltpu.sync_copy(zbuf, acc.at[pl.ds(row0 + z * ZROWS, ZROWS)])
        return 0
    lax.fori_loop(0, STRIPE // ZROWS, _zfill, 0)

    @pl.when(s == 0)
    def _():
        pltpu.sync_copy(zbuf, acc.at[pl.ds(NUM_SUBCORES * STRIPE, TAIL_ROWS)])
    plsc.subcore_barrier()

    e_base = s * EDGES_PER_TILE

    def _group(g, _):
        base = e_base + g * (NBUF * CHUNK)
        lds = []
        for b in range(NBUF):
            e0 = base + b * CHUNK
            lds.append((
                pltpu.async_copy(src_hbm.at[pl.ds(e0, CHUNK)],
                                 sidx.at[b], sems[b]),
                pltpu.async_copy(dst_hbm.at[pl.ds(e0, CHUNK)],
                                 didx.at[b], sems[b]),
                pltpu.async_copy(
                    w_hbm.at[pl.ds(e0 * F_DIM, CHUNK * F_DIM)],
                    wbuf.at[b], sems[b]),
            ))
        gds = []
        for b in range(NBUF):
            for d in lds[b]:
                d.wait()
            for j in range(CHUNK // F_DIM):
                sl = pl.ds(j * F_DIM, F_DIM)
                gidx[b, sl] = sidx[b, sl] * 2 + c
            gds.append(pltpu.async_copy(
                x2_hbm.at[gidx.at[b]], rows.at[b], sems[b]))
        sds = []
        for b in range(NBUF):
            gds[b].wait()

            def _edge(e, _, b=b):
                wv = wbuf[b, pl.ds(e * F_DIM + c * 8, F_DIM)]
                for t in range(T_HALF):
                    rows[b, e, pl.ds(t * F_DIM, F_DIM)] = (
                        rows[b, e, pl.ds(t * F_DIM, F_DIM)] * wv[t])
                return 0
            lax.fori_loop(0, CHUNK, _edge, 0, unroll=2)
            sds.append(pltpu.async_copy(
                rows.at[b], acc.at[didx.at[b]], sems[b], add=True))
        for d in sds:
            d.wait()
        return 0

    lax.fori_loop(0, GROUPS_PER_TILE, _group, 0)
    plsc.subcore_barrier()

    tail0 = NUM_SUBCORES * STRIPE

    @pl.when(c == 0)
    def _():
        pltpu.sync_copy(acc.at[pl.ds(row0, STRIPE)],
                        p0_hbm.at[pl.ds(row0, STRIPE)])

        @pl.when(s == 0)
        def _():
            pltpu.sync_copy(acc.at[pl.ds(tail0, TAIL_ROWS)],
                            p0_hbm.at[pl.ds(tail0, TAIL_ROWS)])

    @pl.when(c == 1)
    def _():
        pltpu.sync_copy(acc.at[pl.ds(row0, STRIPE)],
                        p1_hbm.at[pl.ds(row0, STRIPE)])

        @pl.when(s == 0)
        def _():
            pltpu.sync_copy(acc.at[pl.ds(tail0, TAIL_ROWS)],
                            p1_hbm.at[pl.ds(tail0, TAIL_ROWS)])


def _sc_aggregate(x2, src, dst, wpad):
    mesh = plsc.VectorSubcoreMesh(
        core_axis_name="c", subcore_axis_name="s",
        num_cores=NUM_CORES, num_subcores=NUM_SUBCORES)
    f = pl.kernel(
        _sc_body,
        out_type=(
            jax.ShapeDtypeStruct((N_NODES, HROW), jnp.float32),
            jax.ShapeDtypeStruct((N_NODES, HROW), jnp.float32),
        ),
        mesh=mesh,
        scratch_types=[
            pltpu.VMEM((NBUF, CHUNK), jnp.int32),
            pltpu.VMEM((NBUF, CHUNK), jnp.int32),
            pltpu.VMEM((NBUF, CHUNK), jnp.int32),
            pltpu.VMEM((NBUF, CHUNK * F_DIM), jnp.float32),
            pltpu.VMEM((NBUF, CHUNK, HROW), jnp.float32),
            pltpu.VMEM((ZROWS, HROW), jnp.float32),
            pltpu.VMEM_SHARED((N_NODES, HROW), jnp.float32),
        ] + [pltpu.SemaphoreType.DMA] * NBUF,
        compiler_params=pltpu.CompilerParams(use_tc_tiling_on_sc=False),
    )
    return f(x2, src, dst, wpad)


def _tc_linear_body(p0_ref, p1_ref, wt_ref, b_ref, o0_ref, o1_ref):
    wt = wt_ref[...]
    b = b_ref[...]
    o0_ref[...] = (
        jnp.dot(p0_ref[...], wt, preferred_element_type=jnp.float32) + b)
    o1_ref[...] = (
        jnp.dot(p1_ref[...], wt, preferred_element_type=jnp.float32) + b)


def _tc_linear(p0r, p1r, wt, b2):
    rows_total = N_NODES * T_HALF
    blk = 6000
    grid = rows_total // blk
    return pl.pallas_call(
        _tc_linear_body,
        grid=(grid,),
        in_specs=[
            pl.BlockSpec((blk, F_DIM), lambda i: (i, 0)),
            pl.BlockSpec((blk, F_DIM), lambda i: (i, 0)),
            pl.BlockSpec((F_DIM, F_DIM), lambda i: (0, 0)),
            pl.BlockSpec((1, F_DIM), lambda i: (0, 0)),
        ],
        out_specs=[
            pl.BlockSpec((blk, F_DIM), lambda i: (i, 0)),
            pl.BlockSpec((blk, F_DIM), lambda i: (i, 0)),
        ],
        out_shape=[
            jax.ShapeDtypeStruct((rows_total, F_DIM), jnp.float32),
            jax.ShapeDtypeStruct((rows_total, F_DIM), jnp.float32),
        ],
    )(p0r, p1r, wt, b2)


def kernel(node_features, edge_weights, edge_index, W, b):
    x2 = node_features.reshape(N_NODES * 2, HROW)
    src = edge_index[0]
    dst = edge_index[1]
    wpad = jnp.pad(
        edge_weights.T.reshape(N_EDGES, 2, T_HALF),
        ((0, 0), (0, 0), (0, 8 - T_HALF))).reshape(N_EDGES * F_DIM)
    p0, p1 = _sc_aggregate(x2, src, dst, wpad)
    o0, o1 = _tc_linear(
        p0.reshape(N_NODES * T_HALF, F_DIM),
        p1.reshape(N_NODES * T_HALF, F_DIM),
        W.T, b.reshape(1, F_DIM))
    out = jnp.concatenate(
        [o0.reshape(N_NODES, T_HALF, F_DIM),
         o1.reshape(N_NODES, T_HALF, F_DIM)], axis=1)
    return out
